# Initial kernel scaffold; baseline (speedup 1.0000x reference)
import jax
import jax.numpy as jnp
from jax.experimental import pallas as pl


def kernel(x_nhwc, dww, dwb, pww, pwb):
    raise NotImplementedError("write your pallas kernel here")


# speedup vs baseline: 1.2019x; 1.2019x over previous
import functools

import jax
import jax.numpy as jnp
from jax.experimental import pallas as pl
from jax.experimental.pallas import tpu as pltpu


def _hardswish(x):
    return x * jnp.clip(x + 3.0, 0.0, 6.0) * (1.0 / 6.0)


def _kernel(n_pw_chunks, x_ref, bd_ref, dwb_ref, pw_ref, pwb_ref, o_ref, xp_ref):
    Nb, H, WC = x_ref.shape
    n_blocks = bd_ref.shape[0]
    CH = pw_ref.shape[-1]

    xp_ref[:, 0:1, :] = jnp.zeros((Nb, 1, WC), jnp.float32)
    xp_ref[:, H + 1:H + 2, :] = jnp.zeros((Nb, 1, WC), jnp.float32)

    x = x_ref[...].astype(jnp.float32).reshape(Nb * H, WC)

    for blk in range(n_blocks):
        xp_ref[:, 1:H + 1, :] = x.reshape(Nb, H, WC)

        xcat = jnp.concatenate(
            [xp_ref[:, 0:H, :], xp_ref[:, 1:H + 1, :], xp_ref[:, 2:H + 2, :]],
            axis=2,
        ).reshape(Nb * H, 3 * WC)
        y = jnp.dot(xcat, bd_ref[blk], preferred_element_type=jnp.float32)
        y = _hardswish(y + dwb_ref[blk].reshape(1, WC))

        if n_pw_chunks == 1:
            z = jnp.dot(y, pw_ref[blk], preferred_element_type=jnp.float32)
        else:
            z = jnp.concatenate(
                [
                    jnp.dot(y[:, k * CH:(k + 1) * CH], pw_ref[blk],
                            preferred_element_type=jnp.float32)
                    for k in range(n_pw_chunks)
                ],
                axis=1,
            )
        z = _hardswish(z + pwb_ref[blk].reshape(1, WC))

        x = z + x

    o_ref[...] = x.reshape(Nb, H, WC).astype(o_ref.dtype)


def _build_params(dww, dwb, pww, pwb, W):
    n, _, _, C = dww.shape
    WC = W * C
    eye_c = jnp.eye(C, dtype=dww.dtype)
    bd = jnp.zeros((n, 3, WC, WC), dww.dtype)
    for kw in range(3):
        s = jnp.eye(W, W, 1 - kw, dtype=dww.dtype)
        bd = bd + jnp.einsum(
            "vw,bkc,dc->bkvdwc", s, dww[:, :, kw, :], eye_c
        ).reshape(n, 3, WC, WC)
    bd = bd.reshape(n, 3 * WC, WC)
    dwb_f = jnp.tile(dwb, (1, W))
    pwb_f = jnp.tile(pwb, (1, W))
    ch = 256 if WC % 256 == 0 else WC
    reps = ch // C
    pw_c = jnp.einsum(
        "uv,bio->buivo", jnp.eye(reps, dtype=pww.dtype), pww
    ).reshape(n, ch, ch)
    return bd, dwb_f, pw_c, pwb_f


def kernel(x_nhwc, dww, dwb, pww, pwb):
    N, H, W, C = x_nhwc.shape
    n = dww.shape[0]
    WC = W * C

    bd, dwb_f, pw_c, pwb_f = _build_params(dww, dwb, pww, pwb, W)
    ch = pw_c.shape[-1]
    n_pw_chunks = WC // ch
    x_f = x_nhwc.reshape(N, H, WC)

    Nb = next(nb for nb in (16, 8, 4, 2, 1) if N % nb == 0)

    out = pl.pallas_call(
        functools.partial(_kernel, n_pw_chunks),
        out_shape=jax.ShapeDtypeStruct((N, H, WC), x_nhwc.dtype),
        grid_spec=pltpu.PrefetchScalarGridSpec(
            num_scalar_prefetch=0,
            grid=(N // Nb,),
            in_specs=[
                pl.BlockSpec((Nb, H, WC), lambda b: (b, 0, 0)),
                pl.BlockSpec((n, 3 * WC, WC), lambda b: (0, 0, 0)),
                pl.BlockSpec((n, WC), lambda b: (0, 0)),
                pl.BlockSpec((n, ch, ch), lambda b: (0, 0, 0)),
                pl.BlockSpec((n, WC), lambda b: (0, 0)),
            ],
            out_specs=pl.BlockSpec((Nb, H, WC), lambda b: (b, 0, 0)),
            scratch_shapes=[pltpu.VMEM((Nb, H + 2, WC), jnp.float32)],
        ),
        compiler_params=pltpu.CompilerParams(
            dimension_semantics=("parallel",),
            vmem_limit_bytes=64 * 1024 * 1024,
        ),
    )(x_f, bd, dwb_f, pw_c, pwb_f)
    return out.reshape(N, H, W, C)


# speedup vs baseline: 1.7263x; 1.4363x over previous
import functools

import jax
import jax.numpy as jnp
import numpy as np
from jax.experimental import pallas as pl
from jax.experimental.pallas import tpu as pltpu


def _hardswish(x):
    return x * jnp.clip(x + 3.0, 0.0, 6.0) * (1.0 / 6.0)


def _kernel(C, n_pw_chunks, x_ref, bd_ref, cw_ref, dwb_ref, pw_ref, pwb_ref,
            o_ref, xp_ref):
    Nb, H, WC = x_ref.shape
    n_blocks = bd_ref.shape[0]
    CH = pw_ref.shape[-1]

    xp_ref[:, 0:1, :] = jnp.zeros((Nb, 1, WC), jnp.float32)
    xp_ref[:, H + 1:H + 2, :] = jnp.zeros((Nb, 1, WC), jnp.float32)

    x = x_ref[...].astype(jnp.float32).reshape(Nb * H, WC)

    for blk in range(n_blocks):
        xp_ref[:, 1:H + 1, :] = x.reshape(Nb, H, WC)

        a = xp_ref[:, 0:H, :].reshape(Nb * H, WC)
        c = xp_ref[:, 2:H + 2, :].reshape(Nb * H, WC)
        y = (jnp.dot(a, bd_ref[blk, 0], preferred_element_type=jnp.float32)
             + jnp.dot(c, bd_ref[blk, 1], preferred_element_type=jnp.float32))

        xr = pltpu.roll(x, C, axis=1)
        xl = pltpu.roll(x, WC - C, axis=1)
        y = y + xr * cw_ref[blk, 0].reshape(1, WC)
        y = y + x * cw_ref[blk, 1].reshape(1, WC)
        y = y + xl * cw_ref[blk, 2].reshape(1, WC)

        y = _hardswish(y + dwb_ref[blk].reshape(1, WC))

        if n_pw_chunks == 1:
            z = jnp.dot(y, pw_ref[blk], preferred_element_type=jnp.float32)
        else:
            z = jnp.concatenate(
                [
                    jnp.dot(y[:, k * CH:(k + 1) * CH], pw_ref[blk],
                            preferred_element_type=jnp.float32)
                    for k in range(n_pw_chunks)
                ],
                axis=1,
            )
        z = _hardswish(z + pwb_ref[blk].reshape(1, WC))

        x = z + x

    o_ref[...] = x.reshape(Nb, H, WC).astype(o_ref.dtype)


def _band_masks(W, C):
    WC = W * C
    masks = np.zeros((3, WC, WC), np.float32)
    eye_c = np.eye(C, dtype=np.float32)
    for kw in range(3):
        for w in range(W):
            v = w + kw - 1
            if 0 <= v < W:
                masks[kw, v * C:(v + 1) * C, w * C:(w + 1) * C] = eye_c
    return masks


def _pw_mask(reps, C):
    m = np.zeros((reps * C, reps * C), np.float32)
    for u in range(reps):
        m[u * C:(u + 1) * C, u * C:(u + 1) * C] = 1.0
    return m


def _build_params(dww, dwb, pww, pwb, W):
    n, _, _, C = dww.shape
    WC = W * C
    dww_f = jnp.tile(dww[:, :, :, None, :], (1, 1, 1, W, 1)).reshape(n, 3, 3, WC)
    w_idx = np.arange(WC) // C
    edge = np.ones((3, WC), np.float32)
    edge[0] = (w_idx >= 1).astype(np.float32)
    edge[2] = (w_idx <= W - 2).astype(np.float32)
    dww_f = dww_f * edge[None, None]
    masks = _band_masks(W, C)
    dww_raw = jnp.tile(dww[:, :, :, None, :], (1, 1, 1, W, 1)).reshape(n, 3, 3, WC)
    bd = sum(
        masks[kw][None, None] * dww_raw[:, ::2, kw, None, :]
        for kw in range(3)
    )
    cw = dww_f[:, 1]
    dwb_f = jnp.tile(dwb, (1, W))
    pwb_f = jnp.tile(pwb, (1, W))
    ch = 256 if WC % 256 == 0 else WC
    reps = ch // C
    pw_c = _pw_mask(reps, C) * jnp.tile(pww, (1, reps, reps))
    return bd, cw, dwb_f, pw_c, pwb_f


def kernel(x_nhwc, dww, dwb, pww, pwb):
    N, H, W, C = x_nhwc.shape
    n = dww.shape[0]
    WC = W * C

    bd, cw, dwb_f, pw_c, pwb_f = _build_params(dww, dwb, pww, pwb, W)
    ch = pw_c.shape[-1]
    n_pw_chunks = WC // ch
    x_f = x_nhwc.reshape(N, H, WC)

    Nb = next(nb for nb in (16, 8, 4, 2, 1) if N % nb == 0)

    out = pl.pallas_call(
        functools.partial(_kernel, C, n_pw_chunks),
        out_shape=jax.ShapeDtypeStruct((N, H, WC), x_nhwc.dtype),
        grid_spec=pltpu.PrefetchScalarGridSpec(
            num_scalar_prefetch=0,
            grid=(N // Nb,),
            in_specs=[
                pl.BlockSpec((Nb, H, WC), lambda b: (b, 0, 0)),
                pl.BlockSpec((n, 2, WC, WC), lambda b: (0, 0, 0, 0)),
                pl.BlockSpec((n, 3, WC), lambda b: (0, 0, 0)),
                pl.BlockSpec((n, WC), lambda b: (0, 0)),
                pl.BlockSpec((n, ch, ch), lambda b: (0, 0, 0)),
                pl.BlockSpec((n, WC), lambda b: (0, 0)),
            ],
            out_specs=pl.BlockSpec((Nb, H, WC), lambda b: (b, 0, 0)),
            scratch_shapes=[pltpu.VMEM((Nb, H + 2, WC), jnp.float32)],
        ),
        compiler_params=pltpu.CompilerParams(
            dimension_semantics=("parallel",),
            vmem_limit_bytes=64 * 1024 * 1024,
        ),
    )(x_f, bd, cw, dwb_f, pw_c, pwb_f)
    return out.reshape(N, H, W, C)
